# baseline (speedup 1.0000x reference)
import functools

import jax
import jax.numpy as jnp
from jax.experimental import pallas as pl
from jax.experimental.pallas import tpu as pltpu

_D = 1024
_E = 16
_K = 8
_F = 2 * _D
_FC = 2048
_NF = _F // _FC


def _gate_body(x_ref, w1_ref, b1_ref, w2_ref, b2_ref,
               probs_ref, idx_ref, gates_ref):
    x = x_ref[...]
    h = jnp.maximum(
        jnp.dot(x, w1_ref[...], preferred_element_type=jnp.float32)
        + b1_ref[...], 0.0)
    s = jnp.dot(h, w2_ref[...], preferred_element_type=jnp.float32) \
        + b2_ref[...]
    m = jnp.max(s, axis=1, keepdims=True)
    e = jnp.exp(s - m)
    probs = e / jnp.sum(e, axis=1, keepdims=True)
    probs_ref[...] = probs

    iota_e = jax.lax.broadcasted_iota(jnp.int32, (1, _E), 1)
    iota_k = jax.lax.broadcasted_iota(jnp.int32, (1, _K), 1)
    p = probs
    vals = jnp.zeros((1, _K), jnp.float32)
    idxs = jnp.zeros((1, _K), jnp.int32)
    for i in range(_K):
        mv = jnp.max(p, axis=1, keepdims=True)
        ai = jnp.min(jnp.where(p == mv, iota_e, _E), axis=1, keepdims=True)
        vals = jnp.where(iota_k == i, mv, vals)
        idxs = jnp.where(iota_k == i, ai, idxs)
        p = jnp.where(iota_e == ai, -jnp.inf, p)
    idx_ref[...] = idxs
    vm = jnp.max(vals, axis=1, keepdims=True)
    ev = jnp.exp(vals - vm)
    gates_ref[...] = ev / jnp.sum(ev, axis=1, keepdims=True)


def _expert_body(idx_ref, gate_ref, x_ref, w1_ref, b1_ref, w2_ref, b2_ref,
                 lw_ref, lb_ref, out_ref, acc_ref):
    k = pl.program_id(0)
    f = pl.program_id(1)
    x = x_ref[...]
    hh = jnp.dot(x, w1_ref[0], preferred_element_type=jnp.float32) \
        + b1_ref[0]
    hh = 0.5 * hh * (1.0 + jax.lax.erf(hh * 0.7071067811865476))
    part = jnp.dot(hh, w2_ref[0], preferred_element_type=jnp.float32)

    @pl.when(f == 0)
    def _():
        acc_ref[...] = part + b2_ref[0]

    @pl.when(f != 0)
    def _():
        acc_ref[...] = acc_ref[...] + part

    @pl.when(f == _NF - 1)
    def _():
        oo = acc_ref[...]
        mu = jnp.mean(oo, axis=1, keepdims=True)
        d = oo - mu
        var = jnp.mean(d * d, axis=1, keepdims=True)
        nn = d * jax.lax.rsqrt(var + 1e-5) * lw_ref[0] + lb_ref[0]
        g = gate_ref[k]
        gated = g * nn

        @pl.when(k == 0)
        def _():
            out_ref[...] = gated

        @pl.when(k != 0)
        def _():
            out_ref[...] = out_ref[...] + gated


@jax.jit
def kernel(features, gate_W1, gate_b1, gate_W2, gate_b2,
           We1, be1, We2, be2, ln_w, ln_b):
    x = features.reshape(-1)[:_D].reshape(1, _D)

    probs, idxs, gates = pl.pallas_call(
        _gate_body,
        out_shape=(
            jax.ShapeDtypeStruct((1, _E), jnp.float32),
            jax.ShapeDtypeStruct((1, _K), jnp.int32),
            jax.ShapeDtypeStruct((1, _K), jnp.float32),
        ),
    )(x, gate_W1, gate_b1.reshape(1, -1), gate_W2, gate_b2.reshape(1, -1))

    grid = (_K, _NF)
    expert = pl.pallas_call(
        _expert_body,
        grid_spec=pltpu.PrefetchScalarGridSpec(
            num_scalar_prefetch=2,
            grid=grid,
            in_specs=[
                pl.BlockSpec((1, _D), lambda k, f, idx, g: (0, 0)),
                pl.BlockSpec((1, _D, _FC), lambda k, f, idx, g: (idx[k], 0, f)),
                pl.BlockSpec((1, 1, _FC), lambda k, f, idx, g: (idx[k], 0, f)),
                pl.BlockSpec((1, _FC, _D), lambda k, f, idx, g: (idx[k], f, 0)),
                pl.BlockSpec((1, 1, _D), lambda k, f, idx, g: (idx[k], 0, 0)),
                pl.BlockSpec((1, 1, _D), lambda k, f, idx, g: (idx[k], 0, 0)),
                pl.BlockSpec((1, 1, _D), lambda k, f, idx, g: (idx[k], 0, 0)),
            ],
            out_specs=pl.BlockSpec((1, _D), lambda k, f, idx, g: (0, 0)),
            scratch_shapes=[pltpu.VMEM((1, _D), jnp.float32)],
        ),
        out_shape=jax.ShapeDtypeStruct((1, _D), jnp.float32),
        compiler_params=pltpu.CompilerParams(
            dimension_semantics=("arbitrary", "arbitrary")),
    )(idxs.reshape(_K), gates.reshape(_K), x,
      We1, be1.reshape(_E, 1, _F), We2, be2.reshape(_E, 1, _D),
      ln_w.reshape(_E, 1, _D), ln_b.reshape(_E, 1, _D))

    return expert.reshape(_D), probs.reshape(_E)


# speedup vs baseline: 1.0335x; 1.0335x over previous
import functools

import jax
import jax.numpy as jnp
from jax.experimental import pallas as pl
from jax.experimental.pallas import tpu as pltpu

_D = 1024
_E = 16
_K = 8
_F = 2 * _D
_FC = 1024
_NF = _F // _FC


def _gate_body(x_ref, w1_ref, b1_ref, w2_ref, b2_ref,
               probs_ref, idx_ref, gates_ref):
    x = x_ref[...]
    h = jnp.maximum(
        jnp.dot(x, w1_ref[...], preferred_element_type=jnp.float32)
        + b1_ref[...], 0.0)
    s = jnp.dot(h, w2_ref[...], preferred_element_type=jnp.float32) \
        + b2_ref[...]
    m = jnp.max(s, axis=1, keepdims=True)
    e = jnp.exp(s - m)
    probs = e / jnp.sum(e, axis=1, keepdims=True)
    probs_ref[...] = probs

    iota_e = jax.lax.broadcasted_iota(jnp.int32, (1, _E), 1)
    iota_k = jax.lax.broadcasted_iota(jnp.int32, (1, _K), 1)
    p = probs
    vals = jnp.zeros((1, _K), jnp.float32)
    idxs = jnp.zeros((1, _K), jnp.int32)
    for i in range(_K):
        mv = jnp.max(p, axis=1, keepdims=True)
        ai = jnp.min(jnp.where(p == mv, iota_e, _E), axis=1, keepdims=True)
        vals = jnp.where(iota_k == i, mv, vals)
        idxs = jnp.where(iota_k == i, ai, idxs)
        p = jnp.where(iota_e == ai, -jnp.inf, p)
    idx_ref[...] = idxs
    vm = jnp.max(vals, axis=1, keepdims=True)
    ev = jnp.exp(vals - vm)
    gates_ref[...] = ev / jnp.sum(ev, axis=1, keepdims=True)


def _expert_body(idx_ref, gate_ref, x_ref, w1a_ref, w1b_ref, b1_ref,
                 w2a_ref, w2b_ref, b2_ref, lw_ref, lb_ref, out_ref, acc_ref):
    k = pl.program_id(0)
    f = pl.program_id(1)
    x = x_ref[...]
    b1 = b1_ref[0]
    h = _FC // 2
    hha = jnp.dot(x, w1a_ref[0], preferred_element_type=jnp.float32) \
        + b1[:, :h]
    hhb = jnp.dot(x, w1b_ref[0], preferred_element_type=jnp.float32) \
        + b1[:, h:]
    hha = 0.5 * hha * (1.0 + jax.lax.erf(hha * 0.7071067811865476))
    hhb = 0.5 * hhb * (1.0 + jax.lax.erf(hhb * 0.7071067811865476))
    part = jnp.dot(hha, w2a_ref[0], preferred_element_type=jnp.float32) \
        + jnp.dot(hhb, w2b_ref[0], preferred_element_type=jnp.float32)

    @pl.when(f == 0)
    def _():
        acc_ref[...] = part + b2_ref[0]

    @pl.when(f != 0)
    def _():
        acc_ref[...] = acc_ref[...] + part

    @pl.when(f == _NF - 1)
    def _():
        oo = acc_ref[...]
        mu = jnp.mean(oo, axis=1, keepdims=True)
        d = oo - mu
        var = jnp.mean(d * d, axis=1, keepdims=True)
        nn = d * jax.lax.rsqrt(var + 1e-5) * lw_ref[0] + lb_ref[0]
        g = gate_ref[k]
        gated = g * nn

        @pl.when(k == 0)
        def _():
            out_ref[...] = gated

        @pl.when(k != 0)
        def _():
            out_ref[...] = out_ref[...] + gated


@jax.jit
def kernel(features, gate_W1, gate_b1, gate_W2, gate_b2,
           We1, be1, We2, be2, ln_w, ln_b):
    x = features.reshape(-1)[:_D].reshape(1, _D)

    probs, idxs, gates = pl.pallas_call(
        _gate_body,
        out_shape=(
            jax.ShapeDtypeStruct((1, _E), jnp.float32),
            jax.ShapeDtypeStruct((1, _K), jnp.int32),
            jax.ShapeDtypeStruct((1, _K), jnp.float32),
        ),
    )(x, gate_W1, gate_b1.reshape(1, -1), gate_W2, gate_b2.reshape(1, -1))

    grid = (_K, _NF)
    expert = pl.pallas_call(
        _expert_body,
        grid_spec=pltpu.PrefetchScalarGridSpec(
            num_scalar_prefetch=2,
            grid=grid,
            in_specs=[
                pl.BlockSpec((1, _D), lambda k, f, idx, g: (0, 0)),
                pl.BlockSpec((1, _D, _FC // 2),
                             lambda k, f, idx, g: (idx[k], 0, 2 * f)),
                pl.BlockSpec((1, _D, _FC // 2),
                             lambda k, f, idx, g: (idx[k], 0, 2 * f + 1)),
                pl.BlockSpec((1, 1, _FC), lambda k, f, idx, g: (idx[k], 0, f)),
                pl.BlockSpec((1, _FC // 2, _D),
                             lambda k, f, idx, g: (idx[k], 2 * f, 0)),
                pl.BlockSpec((1, _FC // 2, _D),
                             lambda k, f, idx, g: (idx[k], 2 * f + 1, 0)),
                pl.BlockSpec((1, 1, _D), lambda k, f, idx, g: (idx[k], 0, 0)),
                pl.BlockSpec((1, 1, _D), lambda k, f, idx, g: (idx[k], 0, 0)),
                pl.BlockSpec((1, 1, _D), lambda k, f, idx, g: (idx[k], 0, 0)),
            ],
            out_specs=pl.BlockSpec((1, _D), lambda k, f, idx, g: (0, 0)),
            scratch_shapes=[pltpu.VMEM((1, _D), jnp.float32)],
        ),
        out_shape=jax.ShapeDtypeStruct((1, _D), jnp.float32),
        compiler_params=pltpu.CompilerParams(
            dimension_semantics=("arbitrary", "arbitrary")),
    )(idxs.reshape(_K), gates.reshape(_K), x,
      We1, We1, be1.reshape(_E, 1, _F), We2, We2, be2.reshape(_E, 1, _D),
      ln_w.reshape(_E, 1, _D), ln_b.reshape(_E, 1, _D))

    return expert.reshape(_D), probs.reshape(_E)


# speedup vs baseline: 1.0362x; 1.0026x over previous
import functools

import jax
import jax.numpy as jnp
from jax.experimental import pallas as pl
from jax.experimental.pallas import tpu as pltpu

_D = 1024
_E = 16
_K = 8
_F = 2 * _D
_FC = 2048
_NF = _F // _FC


def _gate_body(x_ref, w1_ref, b1_ref, w2_ref, b2_ref,
               probs_ref, idx_ref, gates_ref):
    x = x_ref[...]
    h = jnp.maximum(
        jnp.dot(x, w1_ref[...], preferred_element_type=jnp.float32)
        + b1_ref[...], 0.0)
    s = jnp.dot(h, w2_ref[...], preferred_element_type=jnp.float32) \
        + b2_ref[...]
    m = jnp.max(s, axis=1, keepdims=True)
    e = jnp.exp(s - m)
    probs = e / jnp.sum(e, axis=1, keepdims=True)
    probs_ref[...] = probs

    iota_e = jax.lax.broadcasted_iota(jnp.int32, (1, _E), 1)
    iota_k = jax.lax.broadcasted_iota(jnp.int32, (1, _K), 1)
    p = probs
    vals = jnp.zeros((1, _K), jnp.float32)
    idxs = jnp.zeros((1, _K), jnp.int32)
    for i in range(_K):
        mv = jnp.max(p, axis=1, keepdims=True)
        ai = jnp.min(jnp.where(p == mv, iota_e, _E), axis=1, keepdims=True)
        vals = jnp.where(iota_k == i, mv, vals)
        idxs = jnp.where(iota_k == i, ai, idxs)
        p = jnp.where(iota_e == ai, -jnp.inf, p)
    idx_ref[...] = idxs
    vm = jnp.max(vals, axis=1, keepdims=True)
    ev = jnp.exp(vals - vm)
    gates_ref[...] = ev / jnp.sum(ev, axis=1, keepdims=True)


def _expert_body(idx_ref, gate_ref, x_ref, w1a_ref, w1b_ref, b1_ref,
                 w2a_ref, w2b_ref, b2_ref, lw_ref, lb_ref, out_ref, acc_ref):
    k = pl.program_id(0)
    f = pl.program_id(1)
    x = x_ref[...]
    b1 = b1_ref[0]
    h = _FC // 2
    hha = jnp.dot(x, w1a_ref[0], preferred_element_type=jnp.float32) \
        + b1[:, :h]
    hhb = jnp.dot(x, w1b_ref[0], preferred_element_type=jnp.float32) \
        + b1[:, h:]
    hha = 0.5 * hha * (1.0 + jax.lax.erf(hha * 0.7071067811865476))
    hhb = 0.5 * hhb * (1.0 + jax.lax.erf(hhb * 0.7071067811865476))
    part = jnp.dot(hha, w2a_ref[0], preferred_element_type=jnp.float32) \
        + jnp.dot(hhb, w2b_ref[0], preferred_element_type=jnp.float32)

    @pl.when(f == 0)
    def _():
        acc_ref[...] = part + b2_ref[0]

    @pl.when(f != 0)
    def _():
        acc_ref[...] = acc_ref[...] + part

    @pl.when(f == _NF - 1)
    def _():
        oo = acc_ref[...]
        mu = jnp.mean(oo, axis=1, keepdims=True)
        d = oo - mu
        var = jnp.mean(d * d, axis=1, keepdims=True)
        nn = d * jax.lax.rsqrt(var + 1e-5) * lw_ref[0] + lb_ref[0]
        g = gate_ref[k]
        gated = g * nn

        @pl.when(k == 0)
        def _():
            out_ref[...] = gated

        @pl.when(k != 0)
        def _():
            out_ref[...] = out_ref[...] + gated


@jax.jit
def kernel(features, gate_W1, gate_b1, gate_W2, gate_b2,
           We1, be1, We2, be2, ln_w, ln_b):
    x = features.reshape(-1)[:_D].reshape(1, _D)

    probs, idxs, gates = pl.pallas_call(
        _gate_body,
        out_shape=(
            jax.ShapeDtypeStruct((1, _E), jnp.float32),
            jax.ShapeDtypeStruct((1, _K), jnp.int32),
            jax.ShapeDtypeStruct((1, _K), jnp.float32),
        ),
    )(x, gate_W1, gate_b1.reshape(1, -1), gate_W2, gate_b2.reshape(1, -1))

    grid = (_K, _NF)
    expert = pl.pallas_call(
        _expert_body,
        grid_spec=pltpu.PrefetchScalarGridSpec(
            num_scalar_prefetch=2,
            grid=grid,
            in_specs=[
                pl.BlockSpec((1, _D), lambda k, f, idx, g: (0, 0)),
                pl.BlockSpec((1, _D, _FC // 2),
                             lambda k, f, idx, g: (idx[k], 0, 2 * f)),
                pl.BlockSpec((1, _D, _FC // 2),
                             lambda k, f, idx, g: (idx[k], 0, 2 * f + 1)),
                pl.BlockSpec((1, 1, _FC), lambda k, f, idx, g: (idx[k], 0, f)),
                pl.BlockSpec((1, _FC // 2, _D),
                             lambda k, f, idx, g: (idx[k], 2 * f, 0)),
                pl.BlockSpec((1, _FC // 2, _D),
                             lambda k, f, idx, g: (idx[k], 2 * f + 1, 0)),
                pl.BlockSpec((1, 1, _D), lambda k, f, idx, g: (idx[k], 0, 0)),
                pl.BlockSpec((1, 1, _D), lambda k, f, idx, g: (idx[k], 0, 0)),
                pl.BlockSpec((1, 1, _D), lambda k, f, idx, g: (idx[k], 0, 0)),
            ],
            out_specs=pl.BlockSpec((1, _D), lambda k, f, idx, g: (0, 0)),
            scratch_shapes=[pltpu.VMEM((1, _D), jnp.float32)],
        ),
        out_shape=jax.ShapeDtypeStruct((1, _D), jnp.float32),
        compiler_params=pltpu.CompilerParams(
            dimension_semantics=("arbitrary", "arbitrary")),
    )(idxs.reshape(_K), gates.reshape(_K), x,
      We1, We1, be1.reshape(_E, 1, _F), We2, We2, be2.reshape(_E, 1, _D),
      ln_w.reshape(_E, 1, _D), ln_b.reshape(_E, 1, _D))

    return expert.reshape(_D), probs.reshape(_E)
